# baseline (speedup 1.0000x reference)
import functools

import jax
import jax.numpy as jnp
from jax import lax
from jax.experimental import pallas as pl
from jax.experimental.pallas import tpu as pltpu
from jax.experimental.pallas import tpu_sc as plsc

EMBED_DIM = 64
NUM_CORES = 2
NUM_SUBCORES = 16
NUM_WORKERS = NUM_CORES * NUM_SUBCORES
CHUNK = 800
NBUF = 2
NSPLIT = 4
SUB = CHUNK // NSPLIT


def _make_call(n_flat):
    assert n_flat % (NUM_WORKERS * CHUNK * NBUF) == 0
    b_per_w = n_flat // NUM_WORKERS
    n_chunks = b_per_w // CHUNK
    mesh = plsc.VectorSubcoreMesh(core_axis_name="c", subcore_axis_name="s")

    @functools.partial(
        pl.kernel,
        mesh=mesh,
        compiler_params=pltpu.CompilerParams(use_tc_tiling_on_sc=False),
        out_type=jax.ShapeDtypeStruct((n_flat, EMBED_DIM), jnp.float32),
        scratch_types=[
            pltpu.VMEM((NBUF, CHUNK), jnp.int32),
            pltpu.VMEM((NBUF, CHUNK, EMBED_DIM), jnp.float32),
            pltpu.SemaphoreType.DMA,
            pltpu.SemaphoreType.DMA,
            pltpu.SemaphoreType.DMA,
        ],
    )
    def gather_kernel(table_hbm, idx_hbm, out_hbm, idx_v, rows_v,
                      idx_sem, gat_sem, out_sem):
        wid = lax.axis_index("s") * NUM_CORES + lax.axis_index("c")
        base = wid * b_per_w

        for b in range(NBUF):
            pltpu.async_copy(
                idx_hbm.at[pl.ds(base + b * CHUNK, CHUNK)], idx_v.at[b],
                idx_sem)

        def outer(g, carry):
            for b in range(NBUF):
                c = g * NBUF + b
                pltpu.make_async_copy(
                    idx_hbm.at[pl.ds(base, CHUNK)], idx_v.at[b],
                    idx_sem).wait()

                @pl.when(g > 0)
                def _():
                    pltpu.make_async_copy(
                        rows_v.at[b], out_hbm.at[pl.ds(base, CHUNK)],
                        out_sem).wait()

                handles = []
                for j in range(NSPLIT):
                    handles.append(pltpu.async_copy(
                        table_hbm.at[idx_v.at[b, pl.ds(j * SUB, SUB)]],
                        rows_v.at[b, pl.ds(j * SUB, SUB)], gat_sem))
                for h in handles:
                    h.wait()

                pltpu.async_copy(
                    rows_v.at[b], out_hbm.at[pl.ds(base + c * CHUNK, CHUNK)],
                    out_sem)

                @pl.when(g < (n_chunks // NBUF) - 1)
                def _():
                    pltpu.async_copy(
                        idx_hbm.at[pl.ds(base + (c + NBUF) * CHUNK, CHUNK)],
                        idx_v.at[b], idx_sem)
            return carry

        lax.fori_loop(0, n_chunks // NBUF, outer, 0)

        for b in range(NBUF):
            pltpu.make_async_copy(
                rows_v.at[b], out_hbm.at[pl.ds(base, CHUNK)], out_sem).wait()

    return gather_kernel


@jax.jit
def kernel(x, table):
    batch, hist = x.shape
    n_flat = batch * hist
    flat_idx = x.reshape(n_flat).astype(jnp.int32)
    out = _make_call(n_flat)(table, flat_idx)
    return out.reshape(batch, hist, EMBED_DIM)


# speedup vs baseline: 1.0027x; 1.0027x over previous
import functools

import jax
import jax.numpy as jnp
from jax import lax
from jax.experimental import pallas as pl
from jax.experimental.pallas import tpu as pltpu
from jax.experimental.pallas import tpu_sc as plsc

EMBED_DIM = 64
NUM_CORES = 2
NUM_SUBCORES = 16
NUM_WORKERS = NUM_CORES * NUM_SUBCORES
NBUF = 2


def _make_call(batch, hist):
    bw = batch // NUM_WORKERS
    assert batch % NUM_WORKERS == 0
    mesh = plsc.VectorSubcoreMesh(core_axis_name="c", subcore_axis_name="s")

    @functools.partial(
        pl.kernel,
        mesh=mesh,
        compiler_params=pltpu.CompilerParams(use_tc_tiling_on_sc=False),
        out_type=jax.ShapeDtypeStruct((batch, hist, EMBED_DIM), jnp.float32),
        scratch_types=[
            pltpu.VMEM((NBUF, bw), jnp.int32),
            pltpu.VMEM((NBUF, bw, EMBED_DIM), jnp.float32),
            pltpu.SemaphoreType.DMA,
            pltpu.SemaphoreType.DMA,
            pltpu.SemaphoreType.DMA,
        ],
    )
    def gather_kernel(table_hbm, xt_hbm, out_hbm, idx_v, rows_v,
                      idx_sem, gat_sem, out_sem):
        wid = lax.axis_index("s") * NUM_CORES + lax.axis_index("c")
        b0 = wid * bw

        for b in range(NBUF):
            pltpu.async_copy(xt_hbm.at[b, pl.ds(b0, bw)], idx_v.at[b],
                             idx_sem)

        def outer(g, carry):
            for b in range(NBUF):
                h = g * NBUF + b
                pltpu.make_async_copy(
                    xt_hbm.at[0, pl.ds(b0, bw)], idx_v.at[b], idx_sem).wait()

                @pl.when(g > 0)
                def _():
                    pltpu.make_async_copy(
                        rows_v.at[b], out_hbm.at[pl.ds(b0, bw), 0],
                        out_sem).wait()

                pltpu.async_copy(
                    table_hbm.at[idx_v.at[b]], rows_v.at[b], gat_sem).wait()

                pltpu.async_copy(
                    rows_v.at[b], out_hbm.at[pl.ds(b0, bw), h], out_sem)

                @pl.when(g < (hist // NBUF) - 1)
                def _():
                    pltpu.async_copy(
                        xt_hbm.at[h + NBUF, pl.ds(b0, bw)], idx_v.at[b],
                        idx_sem)
            return carry

        lax.fori_loop(0, hist // NBUF, outer, 0)

        for b in range(NBUF):
            pltpu.make_async_copy(
                rows_v.at[b], out_hbm.at[pl.ds(b0, bw), 0], out_sem).wait()

    return gather_kernel


@jax.jit
def kernel(x, table):
    batch, hist = x.shape
    xt = x.T.astype(jnp.int32)
    return _make_call(batch, hist)(table, xt)
